# baseline (speedup 1.0000x reference)
import functools

import jax
import jax.numpy as jnp
from jax import lax
from jax.experimental import pallas as pl
from jax.experimental.pallas import tpu as pltpu
from jax.experimental.pallas import tpu_sc as plsc

_SEQ_PER_CHUNK = 8


def _emb_kernel_factory(B, T, D, V, num_cores, num_subcores):
    nw = num_cores * num_subcores
    seq_per_w = B // nw
    spc = _SEQ_PER_CHUNK
    n_chunks = seq_per_w // spc
    half = D // 2

    mesh = plsc.VectorSubcoreMesh(core_axis_name="c", subcore_axis_name="s")

    @functools.partial(
        pl.kernel,
        mesh=mesh,
        compiler_params=pltpu.CompilerParams(use_tc_tiling_on_sc=False),
        out_type=jax.ShapeDtypeStruct((B, T, D), jnp.float32),
        scratch_types=[
            pltpu.VMEM((spc, T), jnp.int32),
            pltpu.VMEM((spc, T, D), jnp.float32),
            pltpu.VMEM((T, D), jnp.float32),
            pltpu.SemaphoreType.DMA,
        ],
    )
    def emb_kernel(x_hbm, tok_hbm, pos_hbm, out_hbm, idx_v, rows_v, pos_v, sem):
        wid = lax.axis_index("s") * num_cores + lax.axis_index("c")
        base = wid * seq_per_w
        pltpu.sync_copy(pos_hbm, pos_v)

        def chunk_body(g, carry):
            s0 = base + g * spc
            pltpu.sync_copy(x_hbm.at[pl.ds(s0, spc)], idx_v)
            copies = [
                pltpu.async_copy(tok_hbm.at[idx_v.at[k]], rows_v.at[k], sem)
                for k in range(spc)
            ]
            for c in copies:
                c.wait()

            def t_body(t, c):
                p0 = pos_v[t, pl.ds(0, half)]
                p1 = pos_v[t, pl.ds(half, half)]
                for k in range(spc):
                    rows_v[k, t, pl.ds(0, half)] += p0
                    rows_v[k, t, pl.ds(half, half)] += p1
                return c

            lax.fori_loop(0, T, t_body, 0)
            pltpu.sync_copy(rows_v, out_hbm.at[pl.ds(s0, spc)])
            return carry

        lax.fori_loop(0, n_chunks, chunk_body, 0)

    return emb_kernel


def kernel(x, token_table, pos_table):
    B, T = x.shape
    V, D = token_table.shape
    info = plsc.get_sparse_core_info()
    emb = _emb_kernel_factory(B, T, D, V, info.num_cores, info.num_subcores)
    return emb(x.astype(jnp.int32), token_table, pos_table)
